# baseline (speedup 1.0000x reference)
import jax
import jax.numpy as jnp
from jax.experimental import pallas as pl
from jax.experimental.pallas import tpu as pltpu

_B = 128
_V = 100000
_S = 8
_NS = _B // _S
_K = 8


def _copy(x_hbm, buf, sems, i):
    k = i % _K
    return pltpu.make_async_copy(
        x_hbm.at[pl.ds(i * _S, _S), :], buf.at[k], sems.at[k]
    )


def _stream_kernel(x_hbm, t_ref, o_ref, buf, sems):
    for i in range(_K):
        _copy(x_hbm, buf, sems, i).start()

    cols = jax.lax.broadcasted_iota(jnp.int32, (_S, _V), 1)
    acc = jnp.float32(0.0)
    for i in range(_NS):
        _copy(x_hbm, buf, sems, i).wait()
        x = buf[i % _K]
        is_t = cols == t_ref[pl.ds(i * _S, _S), :]
        rmax = jnp.maximum(jnp.max(jnp.where(is_t, -jnp.inf, x), axis=1), 0.0)
        tval = jnp.sum(jnp.where(is_t, x, 0.0), axis=1)
        acc = acc + jnp.sum(tval - rmax)
        if i + _K < _NS:
            _copy(x_hbm, buf, sems, i + _K).start()
    o_ref[0, 0] = acc / _B


@jax.jit
def _run(output, t):
    return pl.pallas_call(
        _stream_kernel,
        in_specs=[
            pl.BlockSpec(memory_space=pl.ANY),
            pl.BlockSpec(memory_space=pltpu.MemorySpace.VMEM),
        ],
        out_specs=pl.BlockSpec(memory_space=pltpu.SMEM),
        out_shape=jax.ShapeDtypeStruct((1, 1), jnp.float32),
        scratch_shapes=[
            pltpu.VMEM((_K, _S, _V), jnp.float32),
            pltpu.SemaphoreType.DMA((_K,)),
        ],
    )(output, t)


def kernel(output, targetC):
    t = targetC.astype(jnp.int32).reshape(_B, 1)
    return _run(output, t)[0, 0]


# speedup vs baseline: 1.0749x; 1.0749x over previous
import jax
import jax.numpy as jnp
from jax.experimental import pallas as pl
from jax.experimental.pallas import tpu as pltpu

_B = 128
_V = 100000
_S = 16
_NS = _B // _S
_K = 4


def _copy(x_hbm, buf, sems, i):
    k = i % _K
    return pltpu.make_async_copy(
        x_hbm.at[pl.ds(i * _S, _S), :], buf.at[k], sems.at[k]
    )


def _stream_kernel(x_hbm, t_ref, o_ref, buf, sems):
    for i in range(_K):
        _copy(x_hbm, buf, sems, i).start()

    cols = jax.lax.broadcasted_iota(jnp.int32, (_S, _V), 1)
    acc = jnp.float32(0.0)
    for i in range(_NS):
        _copy(x_hbm, buf, sems, i).wait()
        x = buf[i % _K]
        is_t = cols == t_ref[pl.ds(i * _S, _S), :]
        rmax = jnp.maximum(jnp.max(jnp.where(is_t, -jnp.inf, x), axis=1), 0.0)
        tval = jnp.sum(jnp.where(is_t, x, 0.0), axis=1)
        acc = acc + jnp.sum(tval - rmax)
        if i + _K < _NS:
            _copy(x_hbm, buf, sems, i + _K).start()
    o_ref[0, 0] = acc / _B


@jax.jit
def _run(output, t):
    return pl.pallas_call(
        _stream_kernel,
        in_specs=[
            pl.BlockSpec(memory_space=pl.ANY),
            pl.BlockSpec(memory_space=pltpu.MemorySpace.VMEM),
        ],
        out_specs=pl.BlockSpec(memory_space=pltpu.SMEM),
        out_shape=jax.ShapeDtypeStruct((1, 1), jnp.float32),
        scratch_shapes=[
            pltpu.VMEM((_K, _S, _V), jnp.float32),
            pltpu.SemaphoreType.DMA((_K,)),
        ],
    )(output, t)


def kernel(output, targetC):
    t = targetC.astype(jnp.int32).reshape(_B, 1)
    return _run(output, t)[0, 0]


# speedup vs baseline: 1.0815x; 1.0061x over previous
import jax
import jax.numpy as jnp
from jax.experimental import pallas as pl
from jax.experimental.pallas import tpu as pltpu

_B = 128
_V = 100000
_S = 16
_NS = _B // _S
_K = 4


_H0 = 49920
_H1 = _V - _H0


def _copies(x_hbm, buf0, buf1, sems, i):
    k = i % _K
    rows = pl.ds(i * _S, _S)
    return [
        pltpu.make_async_copy(
            x_hbm.at[rows, pl.ds(0, _H0)], buf0.at[k], sems.at[k, 0]
        ),
        pltpu.make_async_copy(
            x_hbm.at[rows, pl.ds(_H0, _H1)], buf1.at[k], sems.at[k, 1]
        ),
    ]


def _stream_kernel(x_hbm, t_ref, o_ref, buf0, buf1, sems):
    for i in range(_K):
        for c in _copies(x_hbm, buf0, buf1, sems, i):
            c.start()

    cols0 = jax.lax.broadcasted_iota(jnp.int32, (_S, _H0), 1)
    cols1 = jax.lax.broadcasted_iota(jnp.int32, (_S, _H1), 1) + _H0
    acc = jnp.float32(0.0)
    for i in range(_NS):
        for c in _copies(x_hbm, buf0, buf1, sems, i):
            c.wait()
        t = t_ref[pl.ds(i * _S, _S), :]
        x0, x1 = buf0[i % _K], buf1[i % _K]
        is_t0, is_t1 = cols0 == t, cols1 == t
        m0 = jnp.max(jnp.where(is_t0, -jnp.inf, x0), axis=1)
        m1 = jnp.max(jnp.where(is_t1, -jnp.inf, x1), axis=1)
        rmax = jnp.maximum(jnp.maximum(m0, m1), 0.0)
        tval = jnp.sum(jnp.where(is_t0, x0, 0.0), axis=1) + jnp.sum(
            jnp.where(is_t1, x1, 0.0), axis=1
        )
        acc = acc + jnp.sum(tval - rmax)
        if i + _K < _NS:
            for c in _copies(x_hbm, buf0, buf1, sems, i + _K):
                c.start()
    o_ref[0, 0] = acc / _B


@jax.jit
def _run(output, t):
    return pl.pallas_call(
        _stream_kernel,
        in_specs=[
            pl.BlockSpec(memory_space=pl.ANY),
            pl.BlockSpec(memory_space=pltpu.MemorySpace.VMEM),
        ],
        out_specs=pl.BlockSpec(memory_space=pltpu.SMEM),
        out_shape=jax.ShapeDtypeStruct((1, 1), jnp.float32),
        scratch_shapes=[
            pltpu.VMEM((_K, _S, _H0), jnp.float32),
            pltpu.VMEM((_K, _S, _H1), jnp.float32),
            pltpu.SemaphoreType.DMA((_K, 2)),
        ],
    )(output, t)


def kernel(output, targetC):
    t = targetC.astype(jnp.int32).reshape(_B, 1)
    return _run(output, t)[0, 0]
